# Initial kernel scaffold; baseline (speedup 1.0000x reference)
import jax
import jax.numpy as jnp
from jax.experimental import pallas as pl


def kernel(inputs_embeds, token_type_ids, position_ids, pos_table, ln_gamma, ln_beta):
    raise NotImplementedError("write your pallas kernel here")


# speedup vs baseline: 1.3834x; 1.3834x over previous
import functools

import jax
import jax.numpy as jnp
from jax import lax
from jax.experimental import pallas as pl
from jax.experimental.pallas import tpu as pltpu
from jax.experimental.pallas import tpu_sc as plsc

B, S, D = 4, 8192, 1024
N = B * S
EPS = 1e-5

NC, NS = 2, 16
NW = NC * NS
ROWS_PER_W = N // NW
CHUNK = 32


def _sc_gather(table, idx):
    mesh = plsc.VectorSubcoreMesh(core_axis_name="c", subcore_axis_name="s")

    @functools.partial(
        pl.kernel,
        out_type=jax.ShapeDtypeStruct((N, D), jnp.float32),
        mesh=mesh,
        scratch_types=[
            pltpu.VMEM((CHUNK,), jnp.int32),
            pltpu.VMEM((CHUNK, D), jnp.float32),
            pltpu.SemaphoreType.DMA,
        ],
    )
    def k(table_hbm, idx_hbm, out_hbm, idx_v, rows_v, sem):
        wid = lax.axis_index("s") * NC + lax.axis_index("c")
        base = wid * ROWS_PER_W

        @pl.loop(0, ROWS_PER_W, step=CHUNK)
        def _(off):
            pltpu.sync_copy(idx_hbm.at[pl.ds(base + off, CHUNK)], idx_v)
            pltpu.async_copy(table_hbm.at[idx_v], rows_v, sem).wait()
            pltpu.sync_copy(rows_v, out_hbm.at[pl.ds(base + off, CHUNK)])

    return k(table, idx)


def _ln_body(x_ref, p_ref, g_ref, b_ref, o_ref):
    h = x_ref[...] + p_ref[...]
    mean = jnp.mean(h, axis=-1, keepdims=True)
    d = h - mean
    var = jnp.mean(d * d, axis=-1, keepdims=True)
    o_ref[...] = d * lax.rsqrt(var + EPS) * g_ref[...] + b_ref[...]


def _tc_add_ln(x2d, pos, gamma, beta):
    R = 256
    return pl.pallas_call(
        _ln_body,
        grid=(N // R,),
        in_specs=[
            pl.BlockSpec((R, D), lambda i: (i, 0)),
            pl.BlockSpec((R, D), lambda i: (i, 0)),
            pl.BlockSpec((1, D), lambda i: (0, 0)),
            pl.BlockSpec((1, D), lambda i: (0, 0)),
        ],
        out_specs=pl.BlockSpec((R, D), lambda i: (i, 0)),
        out_shape=jax.ShapeDtypeStruct((N, D), jnp.float32),
    )(x2d, pos, gamma, beta)


def kernel(inputs_embeds, token_type_ids, position_ids, pos_table, ln_gamma, ln_beta):
    del token_type_ids
    idx = position_ids.reshape(N).astype(jnp.int32)
    pos = _sc_gather(pos_table, idx)
    x2d = inputs_embeds.reshape(N, D)
    out = _tc_add_ln(x2d, pos, ln_gamma.reshape(1, D), ln_beta.reshape(1, D))
    return out.reshape(B, S, D)


# speedup vs baseline: 1.5164x; 1.0961x over previous
import functools

import jax
import jax.numpy as jnp
from jax import lax
from jax.experimental import pallas as pl
from jax.experimental.pallas import tpu as pltpu
from jax.experimental.pallas import tpu_sc as plsc

B, S, D = 4, 8192, 1024
N = B * S
EPS = 1e-5

NC, NS = 2, 16
NW = NC * NS
ROWS_PER_W = N // NW
CHUNK = 32


def _sc_gather(table, idx):
    mesh = plsc.VectorSubcoreMesh(core_axis_name="c", subcore_axis_name="s")

    @functools.partial(
        pl.kernel,
        out_type=jax.ShapeDtypeStruct((N, D), jnp.float32),
        mesh=mesh,
        scratch_types=[
            pltpu.VMEM((ROWS_PER_W,), jnp.int32),
            pltpu.VMEM((CHUNK, D), jnp.float32),
            pltpu.VMEM((CHUNK, D), jnp.float32),
            pltpu.SemaphoreType.DMA,
            pltpu.SemaphoreType.DMA,
            pltpu.SemaphoreType.DMA,
            pltpu.SemaphoreType.DMA,
        ],
    )
    def k(table_hbm, idx_hbm, out_hbm, idx_v, rv0, rv1, gs0, gs1, ws0, ws1):
        wid = lax.axis_index("s") * NC + lax.axis_index("c")
        base = wid * ROWS_PER_W
        pltpu.sync_copy(idx_hbm.at[pl.ds(base, ROWS_PER_W)], idx_v)

        @pl.loop(0, ROWS_PER_W, step=2 * CHUNK)
        def _(off):
            g0 = pltpu.async_copy(
                table_hbm.at[idx_v.at[pl.ds(off, CHUNK)]], rv0, gs0)
            g1 = pltpu.async_copy(
                table_hbm.at[idx_v.at[pl.ds(off + CHUNK, CHUNK)]], rv1, gs1)
            g0.wait()
            w0 = pltpu.async_copy(rv0, out_hbm.at[pl.ds(base + off, CHUNK)], ws0)
            g1.wait()
            w1 = pltpu.async_copy(
                rv1, out_hbm.at[pl.ds(base + off + CHUNK, CHUNK)], ws1)
            w0.wait()
            w1.wait()

    return k(table, idx)


def _ln_body(x_ref, p_ref, g_ref, b_ref, o_ref):
    h = x_ref[...] + p_ref[...]
    mean = jnp.mean(h, axis=-1, keepdims=True)
    d = h - mean
    var = jnp.mean(d * d, axis=-1, keepdims=True)
    o_ref[...] = d * lax.rsqrt(var + EPS) * g_ref[...] + b_ref[...]


def _tc_add_ln(x2d, pos, gamma, beta):
    R = 256
    return pl.pallas_call(
        _ln_body,
        grid=(N // R,),
        in_specs=[
            pl.BlockSpec((R, D), lambda i: (i, 0)),
            pl.BlockSpec((R, D), lambda i: (i, 0)),
            pl.BlockSpec((1, D), lambda i: (0, 0)),
            pl.BlockSpec((1, D), lambda i: (0, 0)),
        ],
        out_specs=pl.BlockSpec((R, D), lambda i: (i, 0)),
        out_shape=jax.ShapeDtypeStruct((N, D), jnp.float32),
    )(x2d, pos, gamma, beta)


def kernel(inputs_embeds, token_type_ids, position_ids, pos_table, ln_gamma, ln_beta):
    del token_type_ids
    idx = position_ids.reshape(N).astype(jnp.int32)
    pos = _sc_gather(pos_table, idx)
    x2d = inputs_embeds.reshape(N, D)
    out = _tc_add_ln(x2d, pos, ln_gamma.reshape(1, D), ln_beta.reshape(1, D))
    return out.reshape(B, S, D)


# speedup vs baseline: 1.6055x; 1.0587x over previous
import functools

import jax
import jax.numpy as jnp
from jax import lax
from jax.experimental import pallas as pl
from jax.experimental.pallas import tpu as pltpu
from jax.experimental.pallas import tpu_sc as plsc

B, S, D = 4, 8192, 1024
N = B * S
EPS = 1e-5

NCHUNK = 4
CH = N // NCHUNK

NC, NS = 2, 16
NW = NC * NS
ROWS_PER_W = CH // NW
CHUNK = 32

R = 256


def _sc_gather(table, idx_chunk):
    mesh = plsc.VectorSubcoreMesh(core_axis_name="c", subcore_axis_name="s")

    @functools.partial(
        pl.kernel,
        out_type=jax.ShapeDtypeStruct((CH, D), jnp.float32),
        mesh=mesh,
        scratch_types=[
            pltpu.VMEM((ROWS_PER_W,), jnp.int32),
            pltpu.VMEM((CHUNK, D), jnp.float32),
            pltpu.VMEM((CHUNK, D), jnp.float32),
            pltpu.SemaphoreType.DMA,
            pltpu.SemaphoreType.DMA,
            pltpu.SemaphoreType.DMA,
            pltpu.SemaphoreType.DMA,
        ],
    )
    def k(table_hbm, idx_hbm, out_hbm, idx_v, rv0, rv1, gs0, gs1, ws0, ws1):
        wid = lax.axis_index("s") * NC + lax.axis_index("c")
        base = wid * ROWS_PER_W
        pltpu.sync_copy(idx_hbm.at[pl.ds(base, ROWS_PER_W)], idx_v)

        @pl.loop(0, ROWS_PER_W, step=2 * CHUNK)
        def _(off):
            g0 = pltpu.async_copy(
                table_hbm.at[idx_v.at[pl.ds(off, CHUNK)]], rv0, gs0)
            g1 = pltpu.async_copy(
                table_hbm.at[idx_v.at[pl.ds(off + CHUNK, CHUNK)]], rv1, gs1)
            g0.wait()
            w0 = pltpu.async_copy(rv0, out_hbm.at[pl.ds(base + off, CHUNK)], ws0)
            g1.wait()
            w1 = pltpu.async_copy(
                rv1, out_hbm.at[pl.ds(base + off + CHUNK, CHUNK)], ws1)
            w0.wait()
            w1.wait()

    return k(table, idx_chunk)


def _ln_body(prev_ref, x_ref, p_ref, g_ref, b_ref, o_ref):
    del prev_ref
    h = x_ref[...] + p_ref[...]
    mean = jnp.mean(h, axis=-1, keepdims=True)
    d = h - mean
    var = jnp.mean(d * d, axis=-1, keepdims=True)
    o_ref[...] = d * lax.rsqrt(var + EPS) * g_ref[...] + b_ref[...]


def _tc_add_ln_chunk(out_prev, x2d, pos_c, gamma, beta, kc):
    base_blk = kc * (CH // R)
    return pl.pallas_call(
        _ln_body,
        grid=(CH // R,),
        in_specs=[
            pl.BlockSpec(memory_space=pl.ANY),
            pl.BlockSpec((R, D), lambda i: (base_blk + i, 0)),
            pl.BlockSpec((R, D), lambda i: (i, 0)),
            pl.BlockSpec((1, D), lambda i: (0, 0)),
            pl.BlockSpec((1, D), lambda i: (0, 0)),
        ],
        out_specs=pl.BlockSpec((R, D), lambda i: (base_blk + i, 0)),
        out_shape=jax.ShapeDtypeStruct((N, D), jnp.float32),
        input_output_aliases={0: 0},
    )(out_prev, x2d, pos_c, gamma, beta)


def _tc_add_ln_first(x2d, pos_c, gamma, beta):
    return pl.pallas_call(
        lambda x_ref, p_ref, g_ref, b_ref, o_ref: _ln_body(
            None, x_ref, p_ref, g_ref, b_ref, o_ref),
        grid=(CH // R,),
        in_specs=[
            pl.BlockSpec((R, D), lambda i: (i, 0)),
            pl.BlockSpec((R, D), lambda i: (i, 0)),
            pl.BlockSpec((1, D), lambda i: (0, 0)),
            pl.BlockSpec((1, D), lambda i: (0, 0)),
        ],
        out_specs=pl.BlockSpec((R, D), lambda i: (i, 0)),
        out_shape=jax.ShapeDtypeStruct((N, D), jnp.float32),
    )(x2d, pos_c, gamma, beta)


def kernel(inputs_embeds, token_type_ids, position_ids, pos_table, ln_gamma, ln_beta):
    del token_type_ids
    idx = position_ids.reshape(NCHUNK, CH).astype(jnp.int32)
    x2d = inputs_embeds.reshape(N, D)
    g2d = ln_gamma.reshape(1, D)
    b2d = ln_beta.reshape(1, D)

    pos = [_sc_gather(pos_table, idx[k]) for k in range(NCHUNK)]
    out = _tc_add_ln_first(x2d, pos[0], g2d, b2d)
    for k in range(1, NCHUNK):
        out = _tc_add_ln_chunk(out, x2d, pos[k], g2d, b2d, k)
    return out.reshape(B, S, D)


# speedup vs baseline: 1.8092x; 1.1269x over previous
import functools

import jax
import jax.numpy as jnp
from jax import lax
from jax.experimental import pallas as pl
from jax.experimental.pallas import tpu as pltpu
from jax.experimental.pallas import tpu_sc as plsc

B, S, D = 4, 8192, 1024
N = B * S
H = D // 2
EPS = 1e-5

NCHUNK = 4
CH = N // NCHUNK

NC, NS = 2, 16
NW = NC * NS
ROWS_PER_W = CH // NW
CHUNK = 64

R = 256


def _sc_gather(table, idx_chunk):
    mesh = plsc.VectorSubcoreMesh(core_axis_name="c", subcore_axis_name="s")

    @functools.partial(
        pl.kernel,
        out_type=jax.ShapeDtypeStruct((CH, H), jnp.uint32),
        mesh=mesh,
        scratch_types=[
            pltpu.VMEM((ROWS_PER_W,), jnp.int32),
            pltpu.VMEM((CHUNK, H), jnp.uint32),
            pltpu.VMEM((CHUNK, H), jnp.uint32),
            pltpu.SemaphoreType.DMA,
            pltpu.SemaphoreType.DMA,
            pltpu.SemaphoreType.DMA,
            pltpu.SemaphoreType.DMA,
        ],
    )
    def k(table_hbm, idx_hbm, out_hbm, idx_v, rv0, rv1, gs0, gs1, ws0, ws1):
        wid = lax.axis_index("s") * NC + lax.axis_index("c")
        base = wid * ROWS_PER_W
        pltpu.sync_copy(idx_hbm.at[pl.ds(base, ROWS_PER_W)], idx_v)

        @pl.loop(0, ROWS_PER_W, step=2 * CHUNK)
        def _(off):
            g0 = pltpu.async_copy(
                table_hbm.at[idx_v.at[pl.ds(off, CHUNK)]], rv0, gs0)
            g1 = pltpu.async_copy(
                table_hbm.at[idx_v.at[pl.ds(off + CHUNK, CHUNK)]], rv1, gs1)
            g0.wait()
            w0 = pltpu.async_copy(rv0, out_hbm.at[pl.ds(base + off, CHUNK)], ws0)
            g1.wait()
            w1 = pltpu.async_copy(
                rv1, out_hbm.at[pl.ds(base + off + CHUNK, CHUNK)], ws1)
            w0.wait()
            w1.wait()

    return k(table, idx_chunk)


def _ln_body(prev_ref, x_ref, p_ref, g_ref, b_ref, o_ref):
    del prev_ref
    pu = p_ref[...]
    plo = lax.bitcast_convert_type(pu << jnp.uint32(16), jnp.float32)
    phi = lax.bitcast_convert_type(pu & jnp.uint32(0xFFFF0000), jnp.float32)
    x = x_ref[...]
    h0 = x[:, :H] + plo
    h1 = x[:, H:] + phi
    s = jnp.sum(h0, axis=-1, keepdims=True) + jnp.sum(h1, axis=-1, keepdims=True)
    mean = s * (1.0 / D)
    d0 = h0 - mean
    d1 = h1 - mean
    var = (jnp.sum(d0 * d0, axis=-1, keepdims=True)
           + jnp.sum(d1 * d1, axis=-1, keepdims=True)) * (1.0 / D)
    r = lax.rsqrt(var + EPS)
    g = g_ref[...]
    b = b_ref[...]
    o_ref[:, :H] = d0 * r * g[:, :H] + b[:, :H]
    o_ref[:, H:] = d1 * r * g[:, H:] + b[:, H:]


def _tc_add_ln_chunk(out_prev, x2d, pos_c, gamma, beta, kc):
    base_blk = kc * (CH // R)
    return pl.pallas_call(
        _ln_body,
        grid=(CH // R,),
        in_specs=[
            pl.BlockSpec(memory_space=pl.ANY),
            pl.BlockSpec((R, D), lambda i: (base_blk + i, 0)),
            pl.BlockSpec((R, H), lambda i: (i, 0)),
            pl.BlockSpec((1, D), lambda i: (0, 0)),
            pl.BlockSpec((1, D), lambda i: (0, 0)),
        ],
        out_specs=pl.BlockSpec((R, D), lambda i: (base_blk + i, 0)),
        out_shape=jax.ShapeDtypeStruct((N, D), jnp.float32),
        input_output_aliases={0: 0},
    )(out_prev, x2d, pos_c, gamma, beta)


def _tc_add_ln_first(x2d, pos_c, gamma, beta):
    return pl.pallas_call(
        lambda x_ref, p_ref, g_ref, b_ref, o_ref: _ln_body(
            None, x_ref, p_ref, g_ref, b_ref, o_ref),
        grid=(CH // R,),
        in_specs=[
            pl.BlockSpec((R, D), lambda i: (i, 0)),
            pl.BlockSpec((R, H), lambda i: (i, 0)),
            pl.BlockSpec((1, D), lambda i: (0, 0)),
            pl.BlockSpec((1, D), lambda i: (0, 0)),
        ],
        out_specs=pl.BlockSpec((R, D), lambda i: (i, 0)),
        out_shape=jax.ShapeDtypeStruct((N, D), jnp.float32),
    )(x2d, pos_c, gamma, beta)


def _pack_table(pos_table):
    bf = lax.bitcast_convert_type(pos_table.astype(jnp.bfloat16), jnp.uint16)
    lo = bf[:, :H].astype(jnp.uint32)
    hi = bf[:, H:].astype(jnp.uint32)
    return lo | (hi << jnp.uint32(16))


def kernel(inputs_embeds, token_type_ids, position_ids, pos_table, ln_gamma, ln_beta):
    del token_type_ids
    idx = position_ids.reshape(NCHUNK, CH).astype(jnp.int32)
    table_p = _pack_table(pos_table)
    x2d = inputs_embeds.reshape(N, D)
    g2d = ln_gamma.reshape(1, D)
    b2d = ln_beta.reshape(1, D)

    pos = [_sc_gather(table_p, idx[k]) for k in range(NCHUNK)]
    out = _tc_add_ln_first(x2d, pos[0], g2d, b2d)
    for k in range(1, NCHUNK):
        out = _tc_add_ln_chunk(out, x2d, pos[k], g2d, b2d, k)
    return out.reshape(B, S, D)


# speedup vs baseline: 2.0683x; 1.1432x over previous
import functools

import jax
import jax.numpy as jnp
from jax import lax
from jax.experimental import pallas as pl
from jax.experimental.pallas import tpu as pltpu
from jax.experimental.pallas import tpu_sc as plsc

B, S, D = 4, 8192, 1024
N = B * S
H = D // 2
EPS = 1e-5

NCHUNK = 4
CH = N // NCHUNK

NC, NS = 2, 16
NW = NC * NS
ROWS_PER_W = CH // NW
CHUNK = 64

R = 512


def _sc_gather(table, idx_chunk):
    mesh = plsc.VectorSubcoreMesh(core_axis_name="c", subcore_axis_name="s")

    @functools.partial(
        pl.kernel,
        out_type=jax.ShapeDtypeStruct((CH, H), jnp.uint32),
        mesh=mesh,
        scratch_types=[
            pltpu.VMEM((ROWS_PER_W,), jnp.int32),
            pltpu.VMEM((CHUNK, H), jnp.uint32),
            pltpu.VMEM((CHUNK, H), jnp.uint32),
            pltpu.SemaphoreType.DMA,
            pltpu.SemaphoreType.DMA,
            pltpu.SemaphoreType.DMA,
            pltpu.SemaphoreType.DMA,
        ],
    )
    def k(table_hbm, idx_hbm, out_hbm, idx_v, rv0, rv1, gs0, gs1, ws0, ws1):
        wid = lax.axis_index("s") * NC + lax.axis_index("c")
        base = wid * ROWS_PER_W
        pltpu.sync_copy(idx_hbm.at[pl.ds(base, ROWS_PER_W)], idx_v)

        @pl.loop(0, ROWS_PER_W, step=2 * CHUNK)
        def _(off):
            g0 = pltpu.async_copy(
                table_hbm.at[idx_v.at[pl.ds(off, CHUNK)]], rv0, gs0)
            g1 = pltpu.async_copy(
                table_hbm.at[idx_v.at[pl.ds(off + CHUNK, CHUNK)]], rv1, gs1)
            g0.wait()
            w0 = pltpu.async_copy(rv0, out_hbm.at[pl.ds(base + off, CHUNK)], ws0)
            g1.wait()
            w1 = pltpu.async_copy(
                rv1, out_hbm.at[pl.ds(base + off + CHUNK, CHUNK)], ws1)
            w0.wait()
            w1.wait()

    return k(table, idx_chunk)


def _ln_body(prev_ref, x_ref, p_ref, g_ref, b_ref, o_ref):
    del prev_ref
    pu = p_ref[...]
    plo = lax.bitcast_convert_type(pu << jnp.uint32(16), jnp.float32)
    phi = lax.bitcast_convert_type(pu & jnp.uint32(0xFFFF0000), jnp.float32)
    x = x_ref[...]
    h0 = x[:, :H] + plo
    h1 = x[:, H:] + phi
    s = jnp.sum(h0, axis=-1, keepdims=True) + jnp.sum(h1, axis=-1, keepdims=True)
    mean = s * (1.0 / D)
    d0 = h0 - mean
    d1 = h1 - mean
    var = (jnp.sum(d0 * d0, axis=-1, keepdims=True)
           + jnp.sum(d1 * d1, axis=-1, keepdims=True)) * (1.0 / D)
    r = lax.rsqrt(var + EPS)
    g = g_ref[...]
    b = b_ref[...]
    o_ref[:, :H] = d0 * r * g[:, :H] + b[:, :H]
    o_ref[:, H:] = d1 * r * g[:, H:] + b[:, H:]


def _tc_add_ln_chunk(out_prev, x2d, pos_c, gamma, beta, kc):
    base_blk = kc * (CH // R)
    return pl.pallas_call(
        _ln_body,
        grid=(CH // R,),
        in_specs=[
            pl.BlockSpec(memory_space=pl.ANY),
            pl.BlockSpec((R, D), lambda i: (base_blk + i, 0)),
            pl.BlockSpec((R, H), lambda i: (i, 0)),
            pl.BlockSpec((1, D), lambda i: (0, 0)),
            pl.BlockSpec((1, D), lambda i: (0, 0)),
        ],
        out_specs=pl.BlockSpec((R, D), lambda i: (base_blk + i, 0)),
        out_shape=jax.ShapeDtypeStruct((N, D), jnp.float32),
        input_output_aliases={0: 0},
    )(out_prev, x2d, pos_c, gamma, beta)


def _tc_add_ln_first(x2d, pos_c, gamma, beta):
    return pl.pallas_call(
        lambda x_ref, p_ref, g_ref, b_ref, o_ref: _ln_body(
            None, x_ref, p_ref, g_ref, b_ref, o_ref),
        grid=(CH // R,),
        in_specs=[
            pl.BlockSpec((R, D), lambda i: (i, 0)),
            pl.BlockSpec((R, H), lambda i: (i, 0)),
            pl.BlockSpec((1, D), lambda i: (0, 0)),
            pl.BlockSpec((1, D), lambda i: (0, 0)),
        ],
        out_specs=pl.BlockSpec((R, D), lambda i: (i, 0)),
        out_shape=jax.ShapeDtypeStruct((N, D), jnp.float32),
    )(x2d, pos_c, gamma, beta)


def _pack_table(pos_table):
    bf = lax.bitcast_convert_type(pos_table.astype(jnp.bfloat16), jnp.uint16)
    lo = bf[:, :H].astype(jnp.uint32)
    hi = bf[:, H:].astype(jnp.uint32)
    return lo | (hi << jnp.uint32(16))


def kernel(inputs_embeds, token_type_ids, position_ids, pos_table, ln_gamma, ln_beta):
    del token_type_ids
    idx = position_ids.reshape(NCHUNK, CH).astype(jnp.int32)
    table_p = _pack_table(pos_table)
    x2d = inputs_embeds.reshape(N, D)
    g2d = ln_gamma.reshape(1, D)
    b2d = ln_beta.reshape(1, D)

    pos = [_sc_gather(table_p, idx[k]) for k in range(NCHUNK)]
    out = _tc_add_ln_first(x2d, pos[0], g2d, b2d)
    for k in range(1, NCHUNK):
        out = _tc_add_ln_chunk(out, x2d, pos[k], g2d, b2d, k)
    return out.reshape(B, S, D)


# speedup vs baseline: 2.1698x; 1.0491x over previous
import functools

import jax
import jax.numpy as jnp
from jax import lax
from jax.experimental import pallas as pl
from jax.experimental.pallas import tpu as pltpu
from jax.experimental.pallas import tpu_sc as plsc

B, S, D = 4, 8192, 1024
N = B * S
H = D // 2
EPS = 1e-5

NCHUNK = 4
CH = N // NCHUNK

NC, NS = 2, 16
NW = NC * NS
ROWS_PER_W = CH // NW
CHUNK = 64

R = 1024


def _sc_gather(table, idx_chunk):
    mesh = plsc.VectorSubcoreMesh(core_axis_name="c", subcore_axis_name="s")

    @functools.partial(
        pl.kernel,
        out_type=jax.ShapeDtypeStruct((CH, H), jnp.uint32),
        mesh=mesh,
        scratch_types=[
            pltpu.VMEM((ROWS_PER_W,), jnp.int32),
            pltpu.VMEM((CHUNK, H), jnp.uint32),
            pltpu.VMEM((CHUNK, H), jnp.uint32),
            pltpu.SemaphoreType.DMA,
            pltpu.SemaphoreType.DMA,
            pltpu.SemaphoreType.DMA,
            pltpu.SemaphoreType.DMA,
        ],
    )
    def k(table_hbm, idx_hbm, out_hbm, idx_v, rv0, rv1, gs0, gs1, ws0, ws1):
        wid = lax.axis_index("s") * NC + lax.axis_index("c")
        base = wid * ROWS_PER_W
        pltpu.sync_copy(idx_hbm.at[pl.ds(base, ROWS_PER_W)], idx_v)

        @pl.loop(0, ROWS_PER_W, step=2 * CHUNK)
        def _(off):
            g0 = pltpu.async_copy(
                table_hbm.at[idx_v.at[pl.ds(off, CHUNK)]], rv0, gs0)
            g1 = pltpu.async_copy(
                table_hbm.at[idx_v.at[pl.ds(off + CHUNK, CHUNK)]], rv1, gs1)
            g0.wait()
            w0 = pltpu.async_copy(rv0, out_hbm.at[pl.ds(base + off, CHUNK)], ws0)
            g1.wait()
            w1 = pltpu.async_copy(
                rv1, out_hbm.at[pl.ds(base + off + CHUNK, CHUNK)], ws1)
            w0.wait()
            w1.wait()

    return k(table, idx_chunk)


def _ln_body(prev_ref, x_ref, p_ref, g_ref, b_ref, o_ref):
    del prev_ref
    pu = p_ref[...]
    plo = lax.bitcast_convert_type(pu << jnp.uint32(16), jnp.float32)
    phi = lax.bitcast_convert_type(pu & jnp.uint32(0xFFFF0000), jnp.float32)
    x = x_ref[...]
    h0 = x[:, :H] + plo
    h1 = x[:, H:] + phi
    s = jnp.sum(h0, axis=-1, keepdims=True) + jnp.sum(h1, axis=-1, keepdims=True)
    mean = s * (1.0 / D)
    d0 = h0 - mean
    d1 = h1 - mean
    var = (jnp.sum(d0 * d0, axis=-1, keepdims=True)
           + jnp.sum(d1 * d1, axis=-1, keepdims=True)) * (1.0 / D)
    r = lax.rsqrt(var + EPS)
    g = g_ref[...]
    b = b_ref[...]
    o_ref[:, :H] = d0 * r * g[:, :H] + b[:, :H]
    o_ref[:, H:] = d1 * r * g[:, H:] + b[:, H:]


def _tc_add_ln_chunk(out_prev, x2d, pos_c, gamma, beta, kc):
    base_blk = kc * (CH // R)
    return pl.pallas_call(
        _ln_body,
        grid=(CH // R,),
        in_specs=[
            pl.BlockSpec(memory_space=pl.ANY),
            pl.BlockSpec((R, D), lambda i: (base_blk + i, 0)),
            pl.BlockSpec((R, H), lambda i: (i, 0)),
            pl.BlockSpec((1, D), lambda i: (0, 0)),
            pl.BlockSpec((1, D), lambda i: (0, 0)),
        ],
        out_specs=pl.BlockSpec((R, D), lambda i: (base_blk + i, 0)),
        out_shape=jax.ShapeDtypeStruct((N, D), jnp.float32),
        input_output_aliases={0: 0},
    )(out_prev, x2d, pos_c, gamma, beta)


def _tc_add_ln_first(x2d, pos_c, gamma, beta):
    return pl.pallas_call(
        lambda x_ref, p_ref, g_ref, b_ref, o_ref: _ln_body(
            None, x_ref, p_ref, g_ref, b_ref, o_ref),
        grid=(CH // R,),
        in_specs=[
            pl.BlockSpec((R, D), lambda i: (i, 0)),
            pl.BlockSpec((R, H), lambda i: (i, 0)),
            pl.BlockSpec((1, D), lambda i: (0, 0)),
            pl.BlockSpec((1, D), lambda i: (0, 0)),
        ],
        out_specs=pl.BlockSpec((R, D), lambda i: (i, 0)),
        out_shape=jax.ShapeDtypeStruct((N, D), jnp.float32),
    )(x2d, pos_c, gamma, beta)


def _pack_table(pos_table):
    bf = lax.bitcast_convert_type(pos_table.astype(jnp.bfloat16), jnp.uint16)
    lo = bf[:, :H].astype(jnp.uint32)
    hi = bf[:, H:].astype(jnp.uint32)
    return lo | (hi << jnp.uint32(16))


def kernel(inputs_embeds, token_type_ids, position_ids, pos_table, ln_gamma, ln_beta):
    del token_type_ids
    idx = position_ids.reshape(NCHUNK, CH).astype(jnp.int32)
    table_p = _pack_table(pos_table)
    x2d = inputs_embeds.reshape(N, D)
    g2d = ln_gamma.reshape(1, D)
    b2d = ln_beta.reshape(1, D)

    pos = [_sc_gather(table_p, idx[k]) for k in range(NCHUNK)]
    out = _tc_add_ln_first(x2d, pos[0], g2d, b2d)
    for k in range(1, NCHUNK):
        out = _tc_add_ln_chunk(out, x2d, pos[k], g2d, b2d, k)
    return out.reshape(B, S, D)


# speedup vs baseline: 2.1711x; 1.0006x over previous
import functools

import jax
import jax.numpy as jnp
from jax import lax
from jax.experimental import pallas as pl
from jax.experimental.pallas import tpu as pltpu
from jax.experimental.pallas import tpu_sc as plsc

B, S, D = 4, 8192, 1024
N = B * S
H = D // 2
EPS = 1e-5

NCHUNK = 4
CH = N // NCHUNK

NC, NS = 2, 16
NW = NC * NS
ROWS_PER_W = CH // NW
CHUNK = 64

R = 2048


def _sc_gather(table, idx_chunk):
    mesh = plsc.VectorSubcoreMesh(core_axis_name="c", subcore_axis_name="s")

    @functools.partial(
        pl.kernel,
        out_type=jax.ShapeDtypeStruct((CH, H), jnp.uint32),
        mesh=mesh,
        scratch_types=[
            pltpu.VMEM((ROWS_PER_W,), jnp.int32),
            pltpu.VMEM((CHUNK, H), jnp.uint32),
            pltpu.VMEM((CHUNK, H), jnp.uint32),
            pltpu.SemaphoreType.DMA,
            pltpu.SemaphoreType.DMA,
            pltpu.SemaphoreType.DMA,
            pltpu.SemaphoreType.DMA,
        ],
    )
    def k(table_hbm, idx_hbm, out_hbm, idx_v, rv0, rv1, gs0, gs1, ws0, ws1):
        wid = lax.axis_index("s") * NC + lax.axis_index("c")
        base = wid * ROWS_PER_W
        pltpu.sync_copy(idx_hbm.at[pl.ds(base, ROWS_PER_W)], idx_v)

        @pl.loop(0, ROWS_PER_W, step=2 * CHUNK)
        def _(off):
            g0 = pltpu.async_copy(
                table_hbm.at[idx_v.at[pl.ds(off, CHUNK)]], rv0, gs0)
            g1 = pltpu.async_copy(
                table_hbm.at[idx_v.at[pl.ds(off + CHUNK, CHUNK)]], rv1, gs1)
            g0.wait()
            w0 = pltpu.async_copy(rv0, out_hbm.at[pl.ds(base + off, CHUNK)], ws0)
            g1.wait()
            w1 = pltpu.async_copy(
                rv1, out_hbm.at[pl.ds(base + off + CHUNK, CHUNK)], ws1)
            w0.wait()
            w1.wait()

    return k(table, idx_chunk)


def _ln_body(prev_ref, x_ref, p_ref, g_ref, b_ref, o_ref):
    del prev_ref
    pu = p_ref[...]
    plo = lax.bitcast_convert_type(pu << jnp.uint32(16), jnp.float32)
    phi = lax.bitcast_convert_type(pu & jnp.uint32(0xFFFF0000), jnp.float32)
    x = x_ref[...]
    h0 = x[:, :H] + plo
    h1 = x[:, H:] + phi
    s = jnp.sum(h0, axis=-1, keepdims=True) + jnp.sum(h1, axis=-1, keepdims=True)
    mean = s * (1.0 / D)
    d0 = h0 - mean
    d1 = h1 - mean
    var = (jnp.sum(d0 * d0, axis=-1, keepdims=True)
           + jnp.sum(d1 * d1, axis=-1, keepdims=True)) * (1.0 / D)
    r = lax.rsqrt(var + EPS)
    g = g_ref[...]
    b = b_ref[...]
    o_ref[:, :H] = d0 * r * g[:, :H] + b[:, :H]
    o_ref[:, H:] = d1 * r * g[:, H:] + b[:, H:]


def _tc_add_ln_chunk(out_prev, x2d, pos_c, gamma, beta, kc):
    base_blk = kc * (CH // R)
    return pl.pallas_call(
        _ln_body,
        grid=(CH // R,),
        in_specs=[
            pl.BlockSpec(memory_space=pl.ANY),
            pl.BlockSpec((R, D), lambda i: (base_blk + i, 0)),
            pl.BlockSpec((R, H), lambda i: (i, 0)),
            pl.BlockSpec((1, D), lambda i: (0, 0)),
            pl.BlockSpec((1, D), lambda i: (0, 0)),
        ],
        out_specs=pl.BlockSpec((R, D), lambda i: (base_blk + i, 0)),
        out_shape=jax.ShapeDtypeStruct((N, D), jnp.float32),
        input_output_aliases={0: 0},
    )(out_prev, x2d, pos_c, gamma, beta)


def _tc_add_ln_first(x2d, pos_c, gamma, beta):
    return pl.pallas_call(
        lambda x_ref, p_ref, g_ref, b_ref, o_ref: _ln_body(
            None, x_ref, p_ref, g_ref, b_ref, o_ref),
        grid=(CH // R,),
        in_specs=[
            pl.BlockSpec((R, D), lambda i: (i, 0)),
            pl.BlockSpec((R, H), lambda i: (i, 0)),
            pl.BlockSpec((1, D), lambda i: (0, 0)),
            pl.BlockSpec((1, D), lambda i: (0, 0)),
        ],
        out_specs=pl.BlockSpec((R, D), lambda i: (i, 0)),
        out_shape=jax.ShapeDtypeStruct((N, D), jnp.float32),
    )(x2d, pos_c, gamma, beta)


def _pack_table(pos_table):
    bf = lax.bitcast_convert_type(pos_table.astype(jnp.bfloat16), jnp.uint16)
    lo = bf[:, :H].astype(jnp.uint32)
    hi = bf[:, H:].astype(jnp.uint32)
    return lo | (hi << jnp.uint32(16))


def kernel(inputs_embeds, token_type_ids, position_ids, pos_table, ln_gamma, ln_beta):
    del token_type_ids
    idx = position_ids.reshape(NCHUNK, CH).astype(jnp.int32)
    table_p = _pack_table(pos_table)
    x2d = inputs_embeds.reshape(N, D)
    g2d = ln_gamma.reshape(1, D)
    b2d = ln_beta.reshape(1, D)

    pos = [_sc_gather(table_p, idx[k]) for k in range(NCHUNK)]
    out = _tc_add_ln_first(x2d, pos[0], g2d, b2d)
    for k in range(1, NCHUNK):
        out = _tc_add_ln_chunk(out, x2d, pos[k], g2d, b2d, k)
    return out.reshape(B, S, D)
